# baseline (speedup 1.0000x reference)
import functools

import jax
import jax.numpy as jnp
from jax import lax
from jax.experimental import pallas as pl
from jax.experimental.pallas import tpu as pltpu
from jax.experimental.pallas import tpu_sc as plsc

N_NODES = 10000
N_EDGES = 320000
D = 128

NC = 2
NS = 16
NW = NC * NS
E_PER_W = N_EDGES // NW
E_CHUNK = 80
N_CHUNKS = E_PER_W // E_CHUNK
NBUF = 3
ROWS_PER_TILE = 632
N_PAD = ROWS_PER_TILE * NS


def _sc_aggregate(xf, src1, dst3, wts):
    mesh = plsc.VectorSubcoreMesh(core_axis_name="c", subcore_axis_name="s")

    @functools.partial(
        pl.kernel,
        out_type=jax.ShapeDtypeStruct((NC, N_PAD, D), jnp.float32),
        mesh=mesh,
        scratch_types=[
            pltpu.VMEM((NBUF, E_CHUNK), jnp.int32),
            pltpu.VMEM((NBUF, E_CHUNK), jnp.float32),
            pltpu.VMEM((NBUF * E_CHUNK, D), jnp.float32),
            pltpu.VMEM((N_CHUNKS, E_CHUNK), jnp.int32),
            pltpu.VMEM_SHARED((N_PAD, D), jnp.float32),
            [pltpu.SemaphoreType.DMA for _ in range(NBUF)],
            [pltpu.SemaphoreType.DMA for _ in range(NBUF)],
            [pltpu.SemaphoreType.DMA for _ in range(NBUF)],
        ],
    )
    def agg(x_hbm, src_hbm, dst_hbm, w_hbm, out_hbm,
            src_v, w_v, rows, dst_big, acc, sem_i, sem_g, sem_s):
        cid = lax.axis_index("c")
        sid = lax.axis_index("s")
        wid = cid * NS + sid
        ebase = wid * E_PER_W

        stage_cp = pltpu.async_copy(dst_hbm.at[wid], dst_big, sem_i[0])

        zero16 = jnp.zeros((16,), jnp.float32)

        def zero_body(i, carry):
            for j in range(D // 16):
                rows[i, pl.ds(j * 16, 16)] = zero16
            return carry

        lax.fori_loop(0, E_CHUNK, zero_body, 0)
        base = sid * ROWS_PER_TILE
        zview = rows.at[pl.ds(0, E_CHUNK)]
        nfull = ROWS_PER_TILE // E_CHUNK
        zcps = [
            pltpu.async_copy(zview,
                             acc.at[pl.ds(base + t * E_CHUNK, E_CHUNK)],
                             sem_s[0])
            for t in range(nfull)
        ]
        rem = ROWS_PER_TILE - nfull * E_CHUNK
        if rem:
            zcps.append(pltpu.async_copy(
                rows.at[pl.ds(0, rem)],
                acc.at[pl.ds(base + nfull * E_CHUNK, rem)], sem_s[0]))
        for cp in zcps:
            cp.wait()
        stage_cp.wait()
        plsc.subcore_barrier()

        def rows_at(bb):
            return rows.at[pl.ds(bb * E_CHUNK, E_CHUNK)]

        def issue_idx(c, bb):
            pltpu.async_copy(src_hbm.at[pl.ds(ebase + c * E_CHUNK, E_CHUNK)],
                             src_v.at[bb], sem_i[bb])
            pltpu.async_copy(w_hbm.at[pl.ds(ebase + c * E_CHUNK, E_CHUNK)],
                             w_v.at[bb], sem_i[bb])

        def wait_idx(c, bb):
            pltpu.make_async_copy(
                src_hbm.at[pl.ds(ebase + c * E_CHUNK, E_CHUNK)],
                src_v.at[bb], sem_i[bb]).wait()
            pltpu.make_async_copy(
                w_hbm.at[pl.ds(ebase + c * E_CHUNK, E_CHUNK)],
                w_v.at[bb], sem_i[bb]).wait()

        def issue_gather(bb):
            pltpu.async_copy(x_hbm.at[src_v.at[bb]], rows_at(bb), sem_g[bb])

        def wait_gather(bb):
            pltpu.make_async_copy(x_hbm.at[src_v.at[bb]], rows_at(bb),
                                  sem_g[bb]).wait()

        def issue_scatter(c, bb):
            pltpu.async_copy(rows_at(bb), acc.at[dst_big.at[c]], sem_s[bb],
                             add=True)

        def wait_scatter(c, bb):
            pltpu.make_async_copy(rows_at(bb), acc.at[dst_big.at[c]],
                                  sem_s[bb]).wait()

        def multiply(b):
            roff = b * E_CHUNK

            def grp(g, cc):
                wv = w_v[b, pl.ds(g * 16, 16)]
                for l in range(16):
                    w = wv[l]
                    e = roff + g * 16 + l
                    for j in range(D // 16):
                        sl = pl.ds(j * 16, 16)
                        rows[e, sl] = rows[e, sl] * w
                return cc

            lax.fori_loop(0, E_CHUNK // 16, grp, 0)

        def do_chunk(c, p, idx_ahead=True, gather_ahead=True,
                     wait_scat=True):
            b, b1, b2 = p % NBUF, (p + 1) % NBUF, (p + 2) % NBUF
            if idx_ahead:
                issue_idx(c + 2, b2)
            if gather_ahead:
                wait_idx(c + 1, b1)
                if wait_scat:
                    wait_scatter(c - 2, b1)
                issue_gather(b1)
            wait_gather(b)
            multiply(b)
            issue_scatter(c, b)

        issue_idx(0, 0)
        issue_idx(1, 1)
        wait_idx(0, 0)
        issue_gather(0)
        do_chunk(0, 0, wait_scat=False)
        do_chunk(1, 1, wait_scat=False)
        do_chunk(2, 2)

        def triple_body(t, carry):
            c0 = 3 * t
            do_chunk(c0, 0)
            do_chunk(c0 + 1, 1)
            do_chunk(c0 + 2, 2)
            return carry

        lax.fori_loop(1, (N_CHUNKS - 2) // 3, triple_body, 0)
        do_chunk(N_CHUNKS - 2, 0, idx_ahead=False)
        do_chunk(N_CHUNKS - 1, 1, idx_ahead=False, gather_ahead=False)
        for k in (N_CHUNKS - 3, N_CHUNKS - 2, N_CHUNKS - 1):
            wait_scatter(k, k % NBUF)
        plsc.subcore_barrier()

        pltpu.sync_copy(acc.at[pl.ds(base, ROWS_PER_TILE)],
                        out_hbm.at[cid, pl.ds(base, ROWS_PER_TILE), :])

    return agg(xf, src1, dst3, wts)


def _tc_body(p0_ref, p1_ref, w_ref, b_ref, o_ref):
    s = p0_ref[0] + p1_ref[0]
    y = jnp.dot(s, w_ref[...], preferred_element_type=jnp.float32)
    o_ref[...] = jnp.maximum(y + b_ref[...], 0.0)


BLK = 1000


def _tc_finish(partials, W, b2):
    return pl.pallas_call(
        _tc_body,
        grid=(N_NODES // BLK,),
        in_specs=[
            pl.BlockSpec((1, BLK, D), lambda i: (0, i, 0)),
            pl.BlockSpec((1, BLK, D), lambda i: (1, i, 0)),
            pl.BlockSpec((D, D), lambda i: (0, 0)),
            pl.BlockSpec((1, D), lambda i: (0, 0)),
        ],
        out_specs=pl.BlockSpec((BLK, D), lambda i: (i, 0)),
        out_shape=jax.ShapeDtypeStruct((N_NODES, D), jnp.float32),
    )(partials, partials, W, b2)


def kernel(x, edge_index, edge_weight, W, b):
    xf = x.reshape(N_NODES, D)
    ei = edge_index.astype(jnp.int32)
    dst3 = ei[1].reshape(NW, N_CHUNKS, E_CHUNK)
    partials = _sc_aggregate(xf, ei[0], dst3, edge_weight)
    out = _tc_finish(partials, W, b.reshape(1, D))
    return out.reshape(1, N_NODES, D)


# speedup vs baseline: 1.0233x; 1.0233x over previous
import functools

import jax
import jax.numpy as jnp
from jax import lax
from jax.experimental import pallas as pl
from jax.experimental.pallas import tpu as pltpu
from jax.experimental.pallas import tpu_sc as plsc

N_NODES = 10000
N_EDGES = 320000
D = 128

NC = 2
NS = 16
NW = NC * NS
E_PER_W = N_EDGES // NW
E_CHUNK = 80
N_CHUNKS = E_PER_W // E_CHUNK
NBUF = 3
FLUSH_ROWS = 624
FLUSH_TAIL = N_NODES - FLUSH_ROWS * NS


def _sc_aggregate(xf, src1, dst1, wts):
    mesh = plsc.VectorSubcoreMesh(core_axis_name="c", subcore_axis_name="s")

    @functools.partial(
        pl.kernel,
        out_type=jax.ShapeDtypeStruct((NC, N_NODES, D), jnp.float32),
        mesh=mesh,
        scratch_types=[
            pltpu.VMEM((E_PER_W,), jnp.int32),
            pltpu.VMEM((NBUF, E_CHUNK), jnp.float32),
            pltpu.VMEM((NBUF, E_CHUNK), jnp.int32),
            pltpu.VMEM((NBUF * E_CHUNK, D), jnp.float32),
            pltpu.VMEM_SHARED((N_NODES, D), jnp.float32),
            [pltpu.SemaphoreType.DMA for _ in range(NBUF)],
            [pltpu.SemaphoreType.DMA for _ in range(NBUF)],
            [pltpu.SemaphoreType.DMA for _ in range(NBUF)],
        ],
    )
    def agg(x_hbm, src_hbm, dst_hbm, w_hbm, out_hbm,
            src_big, w_v, dst_v, rows, acc, sem_d, sem_g, sem_s):
        cid = lax.axis_index("c")
        sid = lax.axis_index("s")
        wid = cid * NS + sid
        ebase = wid * E_PER_W

        def rows_at(bb):
            return rows.at[pl.ds(bb * E_CHUNK, E_CHUNK)]

        def issue_dw(c, bb):
            pltpu.async_copy(dst_hbm.at[pl.ds(ebase + c * E_CHUNK, E_CHUNK)],
                             dst_v.at[bb], sem_d[bb])
            pltpu.async_copy(w_hbm.at[pl.ds(ebase + c * E_CHUNK, E_CHUNK)],
                             w_v.at[bb], sem_d[bb])

        def wait_dw(c, bb):
            pltpu.make_async_copy(
                dst_hbm.at[pl.ds(ebase + c * E_CHUNK, E_CHUNK)],
                dst_v.at[bb], sem_d[bb]).wait()
            pltpu.make_async_copy(
                w_hbm.at[pl.ds(ebase + c * E_CHUNK, E_CHUNK)],
                w_v.at[bb], sem_d[bb]).wait()

        def src_at(c):
            return src_big.at[pl.ds(c * E_CHUNK, E_CHUNK)]

        def issue_gather(c, bb):
            pltpu.async_copy(x_hbm.at[src_at(c)], rows_at(bb), sem_g[bb])

        def wait_gather(c, bb):
            pltpu.make_async_copy(x_hbm.at[src_at(c)], rows_at(bb),
                                  sem_g[bb]).wait()

        def issue_scatter(c, bb):
            pltpu.async_copy(rows_at(bb), acc.at[dst_v.at[bb]], sem_s[bb],
                             add=True)

        def wait_scatter(c, bb):
            pltpu.make_async_copy(rows_at(bb), acc.at[dst_v.at[bb]],
                                  sem_s[bb]).wait()

        def multiply(c, b):
            roff = b * E_CHUNK

            def grp(g, cc):
                wv = w_v[b, pl.ds(g * 16, 16)]
                for l in range(16):
                    w = wv[l]
                    e = roff + g * 16 + l
                    for j in range(D // 16):
                        sl = pl.ds(j * 16, 16)
                        rows[e, sl] = rows[e, sl] * w
                return cc

            lax.fori_loop(0, E_CHUNK // 16, grp, 0)

        stage_src = pltpu.async_copy(src_hbm.at[pl.ds(ebase, E_PER_W)],
                                     src_big, sem_g[0])
        issue_dw(0, 0)

        zero16 = jnp.zeros((16,), jnp.float32)

        def zero_body(i, carry):
            for j in range(D // 16):
                rows[i, pl.ds(j * 16, 16)] = zero16
            return carry

        lax.fori_loop(0, E_CHUNK, zero_body, 0)
        zbase = sid * FLUSH_ROWS
        zview = rows.at[pl.ds(0, E_CHUNK)]
        nfull = FLUSH_ROWS // E_CHUNK
        zcps = [
            pltpu.async_copy(zview,
                             acc.at[pl.ds(zbase + t * E_CHUNK, E_CHUNK)],
                             sem_s[0])
            for t in range(nfull)
        ]
        zrem = FLUSH_ROWS - nfull * E_CHUNK
        if zrem:
            zcps.append(pltpu.async_copy(
                rows.at[pl.ds(0, zrem)],
                acc.at[pl.ds(zbase + nfull * E_CHUNK, zrem)], sem_s[0]))
        for cp in zcps:
            cp.wait()

        @pl.when(sid == 0)
        def _():
            pltpu.sync_copy(rows.at[pl.ds(0, FLUSH_TAIL)],
                            acc.at[pl.ds(FLUSH_ROWS * NS, FLUSH_TAIL)])

        stage_src.wait()
        plsc.subcore_barrier()

        def do_chunk(c, p, gather_ahead=True, wait_scat=True):
            b, b1 = p % NBUF, (p + 1) % NBUF
            if gather_ahead:
                if wait_scat:
                    wait_scatter(c - 2, b1)
                issue_dw(c + 1, b1)
                issue_gather(c + 1, b1)
            wait_gather(c, b)
            wait_dw(c, b)
            multiply(c, b)
            issue_scatter(c, b)

        issue_gather(0, 0)
        do_chunk(0, 0, wait_scat=False)
        do_chunk(1, 1, wait_scat=False)
        do_chunk(2, 2)

        def triple_body(t, carry):
            c0 = 3 * t
            do_chunk(c0, 0)
            do_chunk(c0 + 1, 1)
            do_chunk(c0 + 2, 2)
            return carry

        lax.fori_loop(1, (N_CHUNKS - 2) // 3, triple_body, 0)
        do_chunk(N_CHUNKS - 2, 0)
        do_chunk(N_CHUNKS - 1, 1, gather_ahead=False)
        for k in (N_CHUNKS - 3, N_CHUNKS - 2, N_CHUNKS - 1):
            wait_scatter(k, k % NBUF)
        plsc.subcore_barrier()

        pltpu.sync_copy(acc.at[pl.ds(zbase, FLUSH_ROWS)],
                        out_hbm.at[cid, pl.ds(zbase, FLUSH_ROWS), :])

        @pl.when(sid == 0)
        def _():
            pltpu.sync_copy(
                acc.at[pl.ds(FLUSH_ROWS * NS, FLUSH_TAIL)],
                out_hbm.at[cid, pl.ds(FLUSH_ROWS * NS, FLUSH_TAIL), :])

    return agg(xf, src1, dst1, wts)


def _tc_body(p0_ref, p1_ref, w_ref, b_ref, o_ref):
    s = p0_ref[0] + p1_ref[0]
    y = jnp.dot(s, w_ref[...], preferred_element_type=jnp.float32)
    o_ref[...] = jnp.maximum(y + b_ref[...], 0.0)


BLK = 1000


def _tc_finish(partials, W, b2):
    return pl.pallas_call(
        _tc_body,
        grid=(N_NODES // BLK,),
        in_specs=[
            pl.BlockSpec((1, BLK, D), lambda i: (0, i, 0)),
            pl.BlockSpec((1, BLK, D), lambda i: (1, i, 0)),
            pl.BlockSpec((D, D), lambda i: (0, 0)),
            pl.BlockSpec((1, D), lambda i: (0, 0)),
        ],
        out_specs=pl.BlockSpec((BLK, D), lambda i: (i, 0)),
        out_shape=jax.ShapeDtypeStruct((N_NODES, D), jnp.float32),
    )(partials, partials, W, b2)


def kernel(x, edge_index, edge_weight, W, b):
    xf = x.reshape(N_NODES, D)
    ei = edge_index.astype(jnp.int32)
    partials = _sc_aggregate(xf, ei[0], ei[1], edge_weight)
    out = _tc_finish(partials, W, b.reshape(1, D))
    return out.reshape(1, N_NODES, D)
